# baseline (device time: 11941 ns/iter reference)
import jax
import jax.numpy as jnp
from jax import lax
from jax.experimental import pallas as pl
from jax.experimental.pallas import tpu as pltpu

N_DEV = 16
EPS = 1e-5


def kernel(x, gamma):
    m, n_local = x.shape
    n_global = N_DEV * n_local
    gamma2d = gamma.reshape(1, n_local)
    assert m % 128 == 0
    ms = m // 128

    def body(
        x_ref, g_ref, out_ref,
        my_ref, planesum_ref, comm_a, comm_b,
        zcredit_sem,
        send_a, recv_a, send_b, recv_b,
    ):
        my_pos = lax.axis_index("i")
        w = lax.rem(my_pos, 4)
        z = my_pos // 4

        barrier_sem = pltpu.get_barrier_semaphore()
        for k in range(1, 4):
            peer_a = z * 4 + lax.rem(w + k, 4)
            pl.semaphore_signal(
                barrier_sem, inc=1,
                device_id=(peer_a,), device_id_type=pl.DeviceIdType.MESH,
            )
            peer_b = lax.rem(z + k, 4) * 4 + w
            pl.semaphore_signal(
                zcredit_sem, inc=1,
                device_id=(peer_b,), device_id_type=pl.DeviceIdType.MESH,
            )

        xx = x_ref[...]
        s_col = jnp.sum(xx * xx, axis=1, keepdims=True)
        eye = (
            lax.broadcasted_iota(jnp.int32, (128, 128), 0)
            == lax.broadcasted_iota(jnp.int32, (128, 128), 1)
        ).astype(jnp.float32)
        packed = jnp.concatenate(
            [
                lax.dot_general(
                    s_col[i * 128 : (i + 1) * 128, :],
                    eye,
                    (((0,), (0,)), ((), ())),
                    preferred_element_type=jnp.float32,
                )
                for i in range(ms)
            ],
            axis=0,
        )
        my_ref[0:ms, :] = packed
        my_ref[ms : 2 * ms, :] = packed

        pl.semaphore_wait(barrier_sem, 3)
        for k in range(1, 4):
            peer_a = z * 4 + lax.rem(w + k, 4)
            rdma = pltpu.make_async_remote_copy(
                src_ref=my_ref,
                dst_ref=comm_a.at[3 - k],
                send_sem=send_a.at[k - 1],
                recv_sem=recv_a.at[3 - k],
                device_id=(peer_a,),
                device_id_type=pl.DeviceIdType.MESH,
            )
            rdma.start()

        xg = xx * g_ref[...]

        for j in range(3):
            recv = pltpu.make_async_remote_copy(
                src_ref=my_ref,
                dst_ref=comm_a.at[j],
                send_sem=send_a.at[j],
                recv_sem=recv_a.at[j],
                device_id=(my_pos,),
                device_id_type=pl.DeviceIdType.MESH,
            )
            recv.wait_recv()

        plane_sum = packed + jnp.sum(comm_a[:, 0:ms, :], axis=0)
        planesum_ref[0:ms, :] = plane_sum
        planesum_ref[ms : 2 * ms, :] = plane_sum

        pl.semaphore_wait(zcredit_sem, 3)
        for k in range(1, 4):
            peer_b = lax.rem(z + k, 4) * 4 + w
            rdma = pltpu.make_async_remote_copy(
                src_ref=planesum_ref,
                dst_ref=comm_b.at[3 - k],
                send_sem=send_b.at[k - 1],
                recv_sem=recv_b.at[3 - k],
                device_id=(peer_b,),
                device_id_type=pl.DeviceIdType.MESH,
            )
            rdma.start()

        for j in range(3):
            recv = pltpu.make_async_remote_copy(
                src_ref=planesum_ref,
                dst_ref=comm_b.at[j],
                send_sem=send_b.at[j],
                recv_sem=recv_b.at[j],
                device_id=(my_pos,),
                device_id_type=pl.DeviceIdType.MESH,
            )
            recv.wait_recv()

        total = plane_sum + jnp.sum(comm_b[:, 0:ms, :], axis=0)
        inv_rms = lax.rsqrt(total * (1.0 / n_global) + EPS)
        inv_col = jnp.concatenate(
            [
                lax.dot_general(
                    eye,
                    inv_rms[i : i + 1, :],
                    (((1,), (1,)), ((), ())),
                    preferred_element_type=jnp.float32,
                )
                for i in range(ms)
            ],
            axis=0,
        )
        out_ref[...] = xg * inv_col

        for k in range(1, 4):
            peer_a = z * 4 + lax.rem(w + k, 4)
            pltpu.make_async_remote_copy(
                src_ref=my_ref,
                dst_ref=comm_a.at[3 - k],
                send_sem=send_a.at[k - 1],
                recv_sem=recv_a.at[3 - k],
                device_id=(peer_a,),
                device_id_type=pl.DeviceIdType.MESH,
            ).wait_send()
            peer_b = lax.rem(z + k, 4) * 4 + w
            pltpu.make_async_remote_copy(
                src_ref=planesum_ref,
                dst_ref=comm_b.at[3 - k],
                send_sem=send_b.at[k - 1],
                recv_sem=recv_b.at[3 - k],
                device_id=(peer_b,),
                device_id_type=pl.DeviceIdType.MESH,
            ).wait_send()

    return pl.pallas_call(
        body,
        out_shape=jax.ShapeDtypeStruct((m, n_local), x.dtype),
        in_specs=[
            pl.BlockSpec(memory_space=pltpu.VMEM),
            pl.BlockSpec(memory_space=pltpu.VMEM),
        ],
        out_specs=pl.BlockSpec(memory_space=pltpu.VMEM),
        scratch_shapes=[
            pltpu.VMEM((2 * ms, 128), jnp.float32),
            pltpu.VMEM((2 * ms, 128), jnp.float32),
            pltpu.VMEM((3, 2 * ms, 128), jnp.float32),
            pltpu.VMEM((3, 2 * ms, 128), jnp.float32),
            pltpu.SemaphoreType.REGULAR,
            pltpu.SemaphoreType.DMA((3,)),
            pltpu.SemaphoreType.DMA((3,)),
            pltpu.SemaphoreType.DMA((3,)),
            pltpu.SemaphoreType.DMA((3,)),
        ],
        compiler_params=pltpu.CompilerParams(collective_id=0),
    )(x, gamma2d)
